# baseline (speedup 1.0000x reference)
import functools
import math

import jax
import jax.numpy as jnp
from jax.experimental import pallas as pl

_H = 16
_BLK = 64
_TOPK_FRAC = 0.5


def _dotb(a, b, dims):
    return jax.lax.dot_general(a.astype(jnp.bfloat16), b.astype(jnp.bfloat16),
                               dims, preferred_element_type=jnp.float32)


def _qkv_kernel(x_ref, w_ref, o_ref):
    o_ref[0] = _dotb(x_ref[...], w_ref[...], (((1,), (0,)), ((), ())))


def _select_kernel(q_ref, k_ref, mask_ref, *, nb, blk, kc, n):
    q = q_ref[0]
    k = k_ref[0]
    row = jax.lax.broadcasted_iota(jnp.int32, (nb, n), 0)
    col = jax.lax.broadcasted_iota(jnp.int32, (nb, n), 1)
    memb = ((col // blk) == row)
    p = jnp.where(memb, 1.0 / blk, 0.0).astype(jnp.float32)
    qb = jnp.dot(p, q, preferred_element_type=jnp.float32,
                 precision=jax.lax.Precision.HIGHEST)
    kb = jnp.dot(p, k, preferred_element_type=jnp.float32,
                 precision=jax.lax.Precision.HIGHEST)
    sim = _dotb(qb, kb, (((1,), (1,)), ((), ())))
    j_iota = jax.lax.broadcasted_iota(jnp.int32, (nb, nb), 1)
    count = jnp.zeros((nb, nb), jnp.float32)
    for jp in range(nb):
        coljp = sim[:, jp:jp + 1]
        beats = (coljp > sim) | ((coljp == sim) & (jp < j_iota))
        count = count + beats.astype(jnp.float32)
    bmask = (count < kc).astype(jnp.float32)
    e = jnp.where(memb, 1.0, 0.0).astype(jnp.float32)
    mask_ref[0] = jnp.dot(bmask, e, preferred_element_type=jnp.float32,
                          precision=jax.lax.Precision.HIGHEST)


def _attn_kernel(q_ref, k_ref, v_ref, m_ref, o_ref, *, scale):
    qi = pl.program_id(1)
    q = q_ref[0]
    k = k_ref[0]
    s = _dotb(q, k, (((1,), (1,)), ((), ()))) * scale
    trow = m_ref[0, pl.ds(qi, 1), :]
    s = jnp.where(trow > 0.0, s, jnp.float32(-1e30))
    mx = jnp.max(s, axis=-1, keepdims=True)
    pexp = jnp.exp(s - mx)
    l = jnp.sum(pexp, axis=-1, keepdims=True)
    o = _dotb(pexp, v_ref[0], (((1,), (0,)), ((), ()))) / l
    o_ref[...] = o


def _proj_kernel(a_ref, w_ref, b_ref, o_ref):
    o_ref[...] = _dotb(a_ref[...], w_ref[...], (((1,), (0,)), ((), ()))) + b_ref[...]


def kernel(x, W_qkv, W_proj, b_proj):
    B, N, C = x.shape
    H = _H
    hd = C // H
    blk = _BLK
    nb = N // blk
    kc = max(1, int(math.ceil(_TOPK_FRAC * nb)))
    scale = hd ** -0.5
    x2 = x.reshape(N, C)

    qkvs = pl.pallas_call(
        _qkv_kernel,
        grid=(3 * H,),
        in_specs=[
            pl.BlockSpec((N, C), lambda j: (0, 0)),
            pl.BlockSpec((C, hd), lambda j: (0, j)),
        ],
        out_specs=pl.BlockSpec((1, N, hd), lambda j: (j, 0, 0)),
        out_shape=jax.ShapeDtypeStruct((3 * H, N, hd), jnp.float32),
    )(x2, W_qkv)

    tmask = pl.pallas_call(
        functools.partial(_select_kernel, nb=nb, blk=blk, kc=kc, n=N),
        grid=(H,),
        in_specs=[
            pl.BlockSpec((1, N, hd), lambda h: (h, 0, 0)),
            pl.BlockSpec((1, N, hd), lambda h: (H + h, 0, 0)),
        ],
        out_specs=pl.BlockSpec((1, nb, N), lambda h: (h, 0, 0)),
        out_shape=jax.ShapeDtypeStruct((H, nb, N), jnp.float32),
    )(qkvs, qkvs)

    attn = pl.pallas_call(
        functools.partial(_attn_kernel, scale=scale),
        grid=(H, nb),
        in_specs=[
            pl.BlockSpec((1, blk, hd), lambda h, qi: (h, qi, 0)),
            pl.BlockSpec((1, N, hd), lambda h, qi: (H + h, 0, 0)),
            pl.BlockSpec((1, N, hd), lambda h, qi: (2 * H + h, 0, 0)),
            pl.BlockSpec((1, nb, N), lambda h, qi: (h, 0, 0)),
        ],
        out_specs=pl.BlockSpec((blk, hd), lambda h, qi: (qi, h)),
        out_shape=jax.ShapeDtypeStruct((N, C), jnp.float32),
    )(qkvs, qkvs, qkvs, tmask)

    bn = 512
    out = pl.pallas_call(
        _proj_kernel,
        grid=(C // bn,),
        in_specs=[
            pl.BlockSpec((N, C), lambda j: (0, 0)),
            pl.BlockSpec((C, bn), lambda j: (0, j)),
            pl.BlockSpec((1, bn), lambda j: (0, j)),
        ],
        out_specs=pl.BlockSpec((N, bn), lambda j: (0, j)),
        out_shape=jax.ShapeDtypeStruct((N, C), jnp.float32),
    )(attn, W_proj, b_proj.reshape(1, C))
    return out.reshape(B, N, C)


# speedup vs baseline: 1.2093x; 1.2093x over previous
import functools
import math

import jax
import jax.numpy as jnp
from jax.experimental import pallas as pl
from jax.experimental.pallas import tpu as pltpu

_H = 16
_BLK = 64
_TOPK_FRAC = 0.5


def _dotb(a, b, dims):
    return jax.lax.dot_general(a.astype(jnp.bfloat16), b.astype(jnp.bfloat16),
                               dims, preferred_element_type=jnp.float32)


def _qkv_kernel(x_ref, w_ref, o_ref, *, hd, heads_per_step):
    r = _dotb(x_ref[...], w_ref[...], (((1,), (0,)), ((), ())))
    for t in range(heads_per_step):
        o_ref[t] = r[:, t * hd:(t + 1) * hd]


def _select_kernel(q_ref, k_ref, idx_ref, *, nb, blk, kc, n):
    q = q_ref[0]
    k = k_ref[0]
    row = jax.lax.broadcasted_iota(jnp.int32, (nb, n), 0)
    col = jax.lax.broadcasted_iota(jnp.int32, (nb, n), 1)
    memb = ((col // blk) == row)
    p = jnp.where(memb, 1.0 / blk, 0.0).astype(jnp.float32)
    qb = jnp.dot(p, q, preferred_element_type=jnp.float32,
                 precision=jax.lax.Precision.HIGHEST)
    kb = jnp.dot(p, k, preferred_element_type=jnp.float32,
                 precision=jax.lax.Precision.HIGHEST)
    sim = _dotb(qb, kb, (((1,), (1,)), ((), ())))
    j_iota = jax.lax.broadcasted_iota(jnp.int32, (nb, nb), 1)
    count = jnp.zeros((nb, nb), jnp.float32)
    for jp in range(nb):
        coljp = sim[:, jp:jp + 1]
        beats = (coljp > sim) | ((coljp == sim) & (jp < j_iota))
        count = count + beats.astype(jnp.float32)
    bmask = (count < kc).astype(jnp.float32)
    lt = (jax.lax.broadcasted_iota(jnp.int32, (nb, nb), 0)
          < jax.lax.broadcasted_iota(jnp.int32, (nb, nb), 1)).astype(jnp.float32)
    pos = jnp.dot(bmask, lt, preferred_element_type=jnp.float32,
                  precision=jax.lax.Precision.HIGHEST)
    p_iota = jax.lax.broadcasted_iota(jnp.int32, (nb, kc, nb), 1).astype(jnp.float32)
    j3 = jax.lax.broadcasted_iota(jnp.int32, (nb, kc, nb), 2).astype(jnp.float32)
    oh = ((pos[:, None, :] == p_iota) & (bmask[:, None, :] > 0)).astype(jnp.float32)
    idxf = jnp.sum(j3 * oh, axis=2)
    idx_ref[0] = idxf.astype(jnp.int32)


def _attn_kernel(idx_ref, q_ref, k_ref, v_ref, o_ref, *, scale, blk, kc):
    h = pl.program_id(0)
    qi = pl.program_id(1)
    q = q_ref[0]
    ks = jnp.concatenate(
        [k_ref[0, pl.ds(idx_ref[h, qi, j] * blk, blk), :] for j in range(kc)], axis=0)
    vs = jnp.concatenate(
        [v_ref[0, pl.ds(idx_ref[h, qi, j] * blk, blk), :] for j in range(kc)], axis=0)
    s = _dotb(q, ks, (((1,), (1,)), ((), ()))) * scale
    mx = jnp.max(s, axis=-1, keepdims=True)
    pexp = jnp.exp(s - mx)
    l = jnp.sum(pexp, axis=-1, keepdims=True)
    o_ref[...] = _dotb(pexp, vs, (((1,), (0,)), ((), ()))) / l


def _proj_kernel(a_ref, w_ref, b_ref, o_ref):
    o_ref[...] = _dotb(a_ref[...], w_ref[...], (((1,), (0,)), ((), ()))) + b_ref[...]


def kernel(x, W_qkv, W_proj, b_proj):
    B, N, C = x.shape
    H = _H
    hd = C // H
    blk = _BLK
    nb = N // blk
    kc = max(1, int(math.ceil(_TOPK_FRAC * nb)))
    scale = hd ** -0.5
    x2 = x.reshape(N, C)

    hps = 4
    qkvs = pl.pallas_call(
        functools.partial(_qkv_kernel, hd=hd, heads_per_step=hps),
        grid=(3 * H // hps,),
        in_specs=[
            pl.BlockSpec((N, C), lambda j: (0, 0)),
            pl.BlockSpec((C, hps * hd), lambda j: (0, j)),
        ],
        out_specs=pl.BlockSpec((hps, N, hd), lambda j: (j, 0, 0)),
        out_shape=jax.ShapeDtypeStruct((3 * H, N, hd), jnp.float32),
    )(x2, W_qkv)

    idx = pl.pallas_call(
        functools.partial(_select_kernel, nb=nb, blk=blk, kc=kc, n=N),
        grid=(H,),
        in_specs=[
            pl.BlockSpec((1, N, hd), lambda h: (h, 0, 0)),
            pl.BlockSpec((1, N, hd), lambda h: (H + h, 0, 0)),
        ],
        out_specs=pl.BlockSpec((1, nb, kc), lambda h: (h, 0, 0)),
        out_shape=jax.ShapeDtypeStruct((H, nb, kc), jnp.int32),
    )(qkvs, qkvs)

    attn = pl.pallas_call(
        functools.partial(_attn_kernel, scale=scale, blk=blk, kc=kc),
        grid=(H, nb),
        in_specs=[
            pl.BlockSpec(memory_space=pltpu.SMEM),
            pl.BlockSpec((1, blk, hd), lambda h, qi: (h, qi, 0)),
            pl.BlockSpec((1, N, hd), lambda h, qi: (H + h, 0, 0)),
            pl.BlockSpec((1, N, hd), lambda h, qi: (2 * H + h, 0, 0)),
        ],
        out_specs=pl.BlockSpec((blk, hd), lambda h, qi: (qi, h)),
        out_shape=jax.ShapeDtypeStruct((N, C), jnp.float32),
    )(idx, qkvs, qkvs, qkvs)

    bn = 512
    out = pl.pallas_call(
        _proj_kernel,
        grid=(C // bn,),
        in_specs=[
            pl.BlockSpec((N, C), lambda j: (0, 0)),
            pl.BlockSpec((C, bn), lambda j: (0, j)),
            pl.BlockSpec((1, bn), lambda j: (0, j)),
        ],
        out_specs=pl.BlockSpec((N, bn), lambda j: (0, j)),
        out_shape=jax.ShapeDtypeStruct((N, C), jnp.float32),
    )(attn, W_proj, b_proj.reshape(1, C))
    return out.reshape(B, N, C)
